# baseline (speedup 1.0000x reference)
import functools

import jax
import jax.numpy as jnp
from jax import lax
from jax.experimental import pallas as pl
from jax.experimental.pallas import tpu as pltpu

T = 4096
HIDDEN = 1024
NH = 16
HS = 64
CHUNK = 64
NHASH = 2
NB = 128
HASH_SEED = 1234
MASKVAL = -1e5
S = NHASH * T
NCH = S // CHUNK


def _k1_body(x_ref, w_ref, o_ref):
    o_ref[0, 0] = jnp.dot(x_ref[0], w_ref[0, 0],
                          preferred_element_type=jnp.float32)


def _project_qkv(ds, hs, W_qk, W_v):
    X2 = jnp.stack([ds, hs])
    W3 = jnp.stack([W_qk, W_qk, W_v])
    W4 = W3.reshape(3, HIDDEN, NH, HS).transpose(0, 2, 1, 3)
    grid = (3, 4, NH)
    return pl.pallas_call(
        _k1_body,
        grid=grid,
        in_specs=[
            pl.BlockSpec((1, 1024, 1024), lambda t, i, h: ((t + 1) // 2, i, 0)),
            pl.BlockSpec((1, 1, 1024, 64), lambda t, i, h: (t, h, 0, 0)),
        ],
        out_specs=pl.BlockSpec((1, 1, 1024, 64), lambda t, i, h: (t, h, i, 0)),
        out_shape=jax.ShapeDtypeStruct((3, NH, T, HS), jnp.float32),
    )(X2, W4)


def _k2_body(x_ref, rot_ref, rank_ref):
    x = x_ref[0, 0]
    rot = rot_ref[0, 0]
    r = jnp.dot(x, rot, preferred_element_type=jnp.float32)

    ii = lax.broadcasted_iota(jnp.int32, (T, HS), 1).astype(jnp.float32)
    mx = jnp.max(r, axis=1, keepdims=True)
    mn = jnp.min(r, axis=1, keepdims=True)
    a1 = jnp.min(jnp.where(r == mx, ii, 64.0), axis=1, keepdims=True)
    a2 = jnp.min(jnp.where(r == mn, ii, 64.0), axis=1, keepdims=True) + 64.0
    bucket = jnp.where(mx >= -mn, a1, a2)

    bi = lax.broadcasted_iota(jnp.int32, (128, NB), 1).astype(jnp.float32)
    rows = lax.broadcasted_iota(jnp.int32, (128, 128), 0)
    cols = lax.broadcasted_iota(jnp.int32, (128, 128), 1)
    lstrict = (rows > cols).astype(jnp.float32)
    ustrict = (rows < cols).astype(jnp.float32)

    run = jnp.zeros((1, NB), jnp.float32)
    bases = []
    for blk in range(32):
        ob = (bucket[blk * 128:(blk + 1) * 128] == bi).astype(jnp.float32)
        bases.append(run)
        run = run + jnp.sum(ob, axis=0, keepdims=True)
    binbase = jnp.dot(run, ustrict, preferred_element_type=jnp.float32)

    for blk in range(32):
        ob = (bucket[blk * 128:(blk + 1) * 128] == bi).astype(jnp.float32)
        pb = jnp.dot(lstrict, ob, preferred_element_type=jnp.float32)
        base = binbase + bases[blk]
        rk = jnp.sum(ob * base, axis=1, keepdims=True) + \
             jnp.sum(ob * pb, axis=1, keepdims=True)
        rank_ref[0, 0, 0, blk * 128:(blk + 1) * 128, :] = rk.astype(jnp.int32)


def _buckets_ranks(qkv, rot4):
    grid = (2, NH, NHASH)
    return pl.pallas_call(
        _k2_body,
        grid=grid,
        in_specs=[
            pl.BlockSpec((1, 1, T, HS), lambda t, h, a: (t, h, 0, 0)),
            pl.BlockSpec((1, 1, HS, HS), lambda t, h, a: (h, a, 0, 0)),
        ],
        out_specs=pl.BlockSpec((1, 1, 1, T, 1), lambda t, h, a: (t, h, a, 0, 0)),
        out_shape=jax.ShapeDtypeStruct((2, NH, NHASH, T, 1), jnp.int32),
    )(qkv, rot4)


def _norm_k(rows):
    var = jnp.mean(rows * rows, axis=1, keepdims=True)
    return rows * lax.rsqrt(var + 1e-6) * (HS ** -0.5)


def _k4_body(qs_ref, ks_ref, vs_ref, pq_ref, pkl_ref,
             kw_ref, vw_ref, pkw_ref,
             out_ref, lg_ref, pk_scr, pv_scr, pp_scr):
    g = pl.program_id(1)

    @pl.when(g == 0)
    def _init():
        pk_scr[...] = _norm_k(kw_ref[0])
        pv_scr[...] = vw_ref[0]
        pp_scr[...] = pkw_ref[0, 0]

    kprev = pk_scr[...]
    vprev = pv_scr[...]
    pprev = pp_scr[...]
    for j in range(16):
        qc = qs_ref[0, j * 64:(j + 1) * 64, :]
        kn = _norm_k(ks_ref[0, j * 64:(j + 1) * 64, :])
        vc = vs_ref[0, j * 64:(j + 1) * 64, :]
        pkc = pkl_ref[0, j, :, :]
        pqc = pq_ref[0, j * 64:(j + 1) * 64, :]
        nt = (((1,), (1,)), ((), ()))
        d0 = lax.dot_general(qc, kprev, nt, preferred_element_type=jnp.float32)
        d1 = lax.dot_general(qc, kn, nt, preferred_element_type=jnp.float32)
        d0 = jnp.where(pqc != pprev, d0, MASKVAL)
        d1 = jnp.where(pqc != pkc, d1, MASKVAL)
        m = jnp.maximum(jnp.max(d0, axis=1, keepdims=True),
                        jnp.max(d1, axis=1, keepdims=True))
        e0 = jnp.exp(d0 - m)
        e1 = jnp.exp(d1 - m)
        s = jnp.sum(e0, axis=1, keepdims=True) + \
            jnp.sum(e1, axis=1, keepdims=True)
        o = jnp.dot(e0, vprev, preferred_element_type=jnp.float32) + \
            jnp.dot(e1, vc, preferred_element_type=jnp.float32)
        out_ref[0, j * 64:(j + 1) * 64, :] = o / s
        lg_ref[0, j * 64:(j + 1) * 64, :] = m + jnp.log(s)
        kprev, vprev, pprev = kn, vc, pkc
    pk_scr[...] = kprev
    pv_scr[...] = vprev
    pp_scr[...] = pprev


def _attention(qs, ks, vs, pq, pkl):
    grid = (NH, 8)
    G = S // 8
    return pl.pallas_call(
        _k4_body,
        grid=grid,
        in_specs=[
            pl.BlockSpec((1, G, HS), lambda h, g: (h, g, 0)),
            pl.BlockSpec((1, G, HS), lambda h, g: (h, g, 0)),
            pl.BlockSpec((1, G, HS), lambda h, g: (h, g, 0)),
            pl.BlockSpec((1, G, 1), lambda h, g: (h, g, 0)),
            pl.BlockSpec((1, 16, 1, HS), lambda h, g: (h, g, 0, 0)),
            pl.BlockSpec((1, 64, HS), lambda h, g: (h, NCH - 1, 0)),
            pl.BlockSpec((1, 64, HS), lambda h, g: (h, NCH - 1, 0)),
            pl.BlockSpec((1, 1, 1, HS), lambda h, g: (h, NCH - 1, 0, 0)),
        ],
        out_specs=[
            pl.BlockSpec((1, G, HS), lambda h, g: (h, g, 0)),
            pl.BlockSpec((1, G, 1), lambda h, g: (h, g, 0)),
        ],
        out_shape=[
            jax.ShapeDtypeStruct((NH, S, HS), jnp.float32),
            jax.ShapeDtypeStruct((NH, S, 1), jnp.float32),
        ],
        scratch_shapes=[
            pltpu.VMEM((64, HS), jnp.float32),
            pltpu.VMEM((64, HS), jnp.float32),
            pltpu.VMEM((1, HS), jnp.float32),
        ],
    )(qs, ks, vs, pq, pkl, ks, vs, pkl)


def _k6_body(ou_ref, lg_ref, f_ref):
    for j in range(2):
        l0 = lg_ref[j, 0]
        l1 = lg_ref[j, 1]
        m = jnp.maximum(l0, l1)
        e0 = jnp.exp(l0 - m)
        e1 = jnp.exp(l1 - m)
        s = e0 + e1
        f_ref[:, j * HS:(j + 1) * HS] = \
            (ou_ref[j, 0] * e0 + ou_ref[j, 1] * e1) / s


def _combine(out_u, lg_u):
    grid = (NH // 2,)
    return pl.pallas_call(
        _k6_body,
        grid=grid,
        in_specs=[
            pl.BlockSpec((2, NHASH, T, HS), lambda g: (g, 0, 0, 0)),
            pl.BlockSpec((2, NHASH, T, 1), lambda g: (g, 0, 0, 0)),
        ],
        out_specs=pl.BlockSpec((T, 2 * HS), lambda g: (0, g)),
        out_shape=jax.ShapeDtypeStruct((T, HIDDEN), jnp.float32),
    )(out_u, lg_u)


def kernel(decoder_states, hidden_states, W_qk, W_v):
    ds = decoder_states[0]
    hs = hidden_states[0]
    qkv = _project_qkv(ds, hs, W_qk, W_v)

    rot = jax.random.normal(jax.random.key(HASH_SEED),
                            (NH, HS, NHASH, NB // 2), jnp.float32)
    rot4 = rot.transpose(0, 2, 1, 3)
    ranks = _buckets_ranks(qkv, rot4)[..., 0]
    rank_q, rank_k = ranks[0], ranks[1]

    posq = jnp.argsort(rank_q, axis=-1).astype(jnp.int32).reshape(NH, S)
    posk = jnp.argsort(rank_k, axis=-1).astype(jnp.int32).reshape(NH, S)
    qs = jnp.take_along_axis(qkv[0], posq[..., None], axis=1)
    ks = jnp.take_along_axis(qkv[1], posk[..., None], axis=1)
    vs = jnp.take_along_axis(qkv[2], posk[..., None], axis=1)

    pq = posq.reshape(NH, S, 1).astype(jnp.float32)
    pkl = posk.reshape(NH, NCH, 1, CHUNK).astype(jnp.float32)
    out_s, lg_s = _attention(qs, ks, vs, pq, pkl)

    rkg = (rank_k + jnp.arange(NHASH, dtype=jnp.int32)[None, :, None] * T)
    rkg = rkg.reshape(NH, S)
    out_u = jnp.take_along_axis(out_s, rkg[..., None], axis=1)
    lg_u = jnp.take_along_axis(lg_s[..., 0], rkg, axis=1)

    final = _combine(out_u.reshape(NH, NHASH, T, HS),
                     lg_u.reshape(NH, NHASH, T, 1))
    return final.reshape(1, T, HIDDEN)


# speedup vs baseline: 4.3993x; 4.3993x over previous
import functools

import jax
import jax.numpy as jnp
from jax import lax
from jax.experimental import pallas as pl
from jax.experimental.pallas import tpu as pltpu
from jax.experimental.pallas import tpu_sc as plsc

T = 4096
HIDDEN = 1024
NH = 16
HS = 64
CHUNK = 64
NHASH = 2
NB = 128
HASH_SEED = 1234
MASKVAL = -1e5
S = NHASH * T
NCH = S // CHUNK


def _k1_body(x_ref, w_ref, o_ref):
    o_ref[0, 0] = jnp.dot(x_ref[0], w_ref[0, 0],
                          preferred_element_type=jnp.float32)


def _project_qkv(ds, hs, W_qk, W_v):
    X2 = jnp.stack([ds, hs])
    W3 = jnp.stack([W_qk, W_qk, W_v])
    W4 = W3.reshape(3, HIDDEN, NH, HS).transpose(0, 2, 1, 3)
    grid = (3, 4, NH)
    return pl.pallas_call(
        _k1_body,
        grid=grid,
        in_specs=[
            pl.BlockSpec((1, 1024, 1024), lambda t, i, h: ((t + 1) // 2, i, 0)),
            pl.BlockSpec((1, 1, 1024, 64), lambda t, i, h: (t, h, 0, 0)),
        ],
        out_specs=pl.BlockSpec((1, 1, 1024, 64), lambda t, i, h: (t, h, i, 0)),
        out_shape=jax.ShapeDtypeStruct((3, NH, T, HS), jnp.float32),
    )(X2, W4)


def _k2_body(x_ref, rot_ref, rank_ref):
    x = x_ref[0, 0]
    rot = rot_ref[0, 0]
    r = jnp.dot(x, rot, preferred_element_type=jnp.float32)

    ii = lax.broadcasted_iota(jnp.int32, (T, HS), 1).astype(jnp.float32)
    mx = jnp.max(r, axis=1, keepdims=True)
    mn = jnp.min(r, axis=1, keepdims=True)
    a1 = jnp.min(jnp.where(r == mx, ii, 64.0), axis=1, keepdims=True)
    a2 = jnp.min(jnp.where(r == mn, ii, 64.0), axis=1, keepdims=True) + 64.0
    bucket = jnp.where(mx >= -mn, a1, a2)

    bi = lax.broadcasted_iota(jnp.int32, (128, NB), 1).astype(jnp.float32)
    rows = lax.broadcasted_iota(jnp.int32, (128, 128), 0)
    cols = lax.broadcasted_iota(jnp.int32, (128, 128), 1)
    lstrict = (rows > cols).astype(jnp.float32)
    ustrict = (rows < cols).astype(jnp.float32)

    run = jnp.zeros((1, NB), jnp.float32)
    bases = []
    for blk in range(32):
        ob = (bucket[blk * 128:(blk + 1) * 128] == bi).astype(jnp.float32)
        bases.append(run)
        run = run + jnp.sum(ob, axis=0, keepdims=True)
    binbase = jnp.dot(run, ustrict, preferred_element_type=jnp.float32)

    for blk in range(32):
        ob = (bucket[blk * 128:(blk + 1) * 128] == bi).astype(jnp.float32)
        pb = jnp.dot(lstrict, ob, preferred_element_type=jnp.float32)
        base = binbase + bases[blk]
        rk = jnp.sum(ob * base, axis=1, keepdims=True) + \
             jnp.sum(ob * pb, axis=1, keepdims=True)
        rank_ref[0, 0, 0, blk * 128:(blk + 1) * 128, :] = rk.astype(jnp.int32)


def _buckets_ranks(qkv, rot4):
    grid = (2, NH, NHASH)
    return pl.pallas_call(
        _k2_body,
        grid=grid,
        in_specs=[
            pl.BlockSpec((1, 1, T, HS), lambda t, h, a: (t, h, 0, 0)),
            pl.BlockSpec((1, 1, HS, HS), lambda t, h, a: (h, a, 0, 0)),
        ],
        out_specs=pl.BlockSpec((1, 1, 1, T, 1), lambda t, h, a: (t, h, a, 0, 0)),
        out_shape=jax.ShapeDtypeStruct((2, NH, NHASH, T, 1), jnp.int32),
    )(qkv, rot4)


def _norm_k(rows):
    var = jnp.mean(rows * rows, axis=1, keepdims=True)
    return rows * lax.rsqrt(var + 1e-6) * (HS ** -0.5)


def _k4_body(qs_ref, ks_ref, vs_ref, pq_ref, pkl_ref,
             kw_ref, vw_ref, pkw_ref,
             out_ref, lg_ref, pk_scr, pv_scr, pp_scr):
    g = pl.program_id(1)

    @pl.when(g == 0)
    def _init():
        pk_scr[...] = _norm_k(kw_ref[0])
        pv_scr[...] = vw_ref[0]
        pp_scr[...] = pkw_ref[0, 0]

    kprev = pk_scr[...]
    vprev = pv_scr[...]
    pprev = pp_scr[...]
    for j in range(16):
        qc = qs_ref[0, j * 64:(j + 1) * 64, :]
        kn = _norm_k(ks_ref[0, j * 64:(j + 1) * 64, :])
        vc = vs_ref[0, j * 64:(j + 1) * 64, :]
        pkc = pkl_ref[0, j, :, :]
        pqc = pq_ref[0, j * 64:(j + 1) * 64, :]
        nt = (((1,), (1,)), ((), ()))
        d0 = lax.dot_general(qc, kprev, nt, preferred_element_type=jnp.float32)
        d1 = lax.dot_general(qc, kn, nt, preferred_element_type=jnp.float32)
        d0 = jnp.where(pqc != pprev, d0, MASKVAL)
        d1 = jnp.where(pqc != pkc, d1, MASKVAL)
        m = jnp.maximum(jnp.max(d0, axis=1, keepdims=True),
                        jnp.max(d1, axis=1, keepdims=True))
        e0 = jnp.exp(d0 - m)
        e1 = jnp.exp(d1 - m)
        s = jnp.sum(e0, axis=1, keepdims=True) + \
            jnp.sum(e1, axis=1, keepdims=True)
        o = jnp.dot(e0, vprev, preferred_element_type=jnp.float32) + \
            jnp.dot(e1, vc, preferred_element_type=jnp.float32)
        out_ref[0, j * 64:(j + 1) * 64, :] = o / s
        lg_ref[0, j * 64:(j + 1) * 64, :] = m + jnp.log(s)
        kprev, vprev, pprev = kn, vc, pkc
    pk_scr[...] = kprev
    pv_scr[...] = vprev
    pp_scr[...] = pprev


def _attention(qs, ks, vs, pq, pkl):
    grid = (NH, 8)
    G = S // 8
    return pl.pallas_call(
        _k4_body,
        grid=grid,
        in_specs=[
            pl.BlockSpec((1, G, HS), lambda h, g: (h, g, 0)),
            pl.BlockSpec((1, G, HS), lambda h, g: (h, g, 0)),
            pl.BlockSpec((1, G, HS), lambda h, g: (h, g, 0)),
            pl.BlockSpec((1, G, 1), lambda h, g: (h, g, 0)),
            pl.BlockSpec((1, 16, 1, HS), lambda h, g: (h, g, 0, 0)),
            pl.BlockSpec((1, 64, HS), lambda h, g: (h, NCH - 1, 0)),
            pl.BlockSpec((1, 64, HS), lambda h, g: (h, NCH - 1, 0)),
            pl.BlockSpec((1, 1, 1, HS), lambda h, g: (h, NCH - 1, 0, 0)),
        ],
        out_specs=[
            pl.BlockSpec((1, G, HS), lambda h, g: (h, g, 0)),
            pl.BlockSpec((1, G, 1), lambda h, g: (h, g, 0)),
        ],
        out_shape=[
            jax.ShapeDtypeStruct((NH, S, HS), jnp.float32),
            jax.ShapeDtypeStruct((NH, S, 1), jnp.float32),
        ],
        scratch_shapes=[
            pltpu.VMEM((64, HS), jnp.float32),
            pltpu.VMEM((64, HS), jnp.float32),
            pltpu.VMEM((1, HS), jnp.float32),
        ],
    )(qs, ks, vs, pq, pkl, ks, vs, pkl)


def _k6_body(ou_ref, lg_ref, f_ref):
    for j in range(2):
        l0 = lg_ref[j, 0]
        l1 = lg_ref[j, 1]
        m = jnp.maximum(l0, l1)
        e0 = jnp.exp(l0 - m)
        e1 = jnp.exp(l1 - m)
        s = e0 + e1
        f_ref[:, j * HS:(j + 1) * HS] = \
            (ou_ref[j, 0] * e0 + ou_ref[j, 1] * e1) / s


def _combine(out_u, lg_u):
    grid = (NH // 2,)
    return pl.pallas_call(
        _k6_body,
        grid=grid,
        in_specs=[
            pl.BlockSpec((2, NHASH, T, HS), lambda g: (g, 0, 0, 0)),
            pl.BlockSpec((2, NHASH, T, 1), lambda g: (g, 0, 0, 0)),
        ],
        out_specs=pl.BlockSpec((T, 2 * HS), lambda g: (0, g)),
        out_shape=jax.ShapeDtypeStruct((T, HIDDEN), jnp.float32),
    )(out_u, lg_u)


_SC_MESH = plsc.VectorSubcoreMesh(core_axis_name="c", subcore_axis_name="s")


def _sc_sort_gather(rank_q2, rank_k2, qtab, ktab, vtab):
    @functools.partial(
        pl.kernel,
        mesh=_SC_MESH,
        compiler_params=pltpu.CompilerParams(needs_layout_passes=False, use_tc_tiling_on_sc=False),
        out_type=[
            jax.ShapeDtypeStruct((32, T), jnp.float32),
            jax.ShapeDtypeStruct((32, T), jnp.float32),
            jax.ShapeDtypeStruct((NH * S, HS), jnp.float32),
            jax.ShapeDtypeStruct((NH * S, HS), jnp.float32),
            jax.ShapeDtypeStruct((NH * S, HS), jnp.float32),
        ],
        scratch_types=[
            pltpu.VMEM((T,), jnp.int32),
            pltpu.VMEM((T,), jnp.int32),
            pltpu.VMEM((T,), jnp.int32),
            pltpu.VMEM((T,), jnp.float32),
            pltpu.VMEM((128, HS), jnp.float32),
            pltpu.SemaphoreType.DMA,
        ],
    )
    def k(rq, rk, qt, kt, vt, posqf, poskf, qs, ks, vs,
          rank_vm, idxq_vm, idxk_vm, posf_vm, rows_vm, sem):
        wid = lax.axis_index("s") * 2 + lax.axis_index("c")
        h = wid // 2
        a = wid % 2

        def invert(rank_hbm, idx_vm, posf_hbm):
            pltpu.sync_copy(rank_hbm.at[wid], rank_vm)

            def inv_step(j, _):
                idx = rank_vm[pl.ds(j * 16, 16)]
                base = j * 16 + lax.iota(jnp.int32, 16)
                plsc.store_scatter(idx_vm, [idx], base + h * T)
                plsc.store_scatter(posf_vm, [idx], base.astype(jnp.float32))
                return 0

            lax.fori_loop(0, T // 16, inv_step, 0)
            pltpu.sync_copy(posf_vm, posf_hbm.at[wid])

        def gather(tab, idx_vm, dst, dst_base):
            def g_step(j, _):
                src = tab.at[idx_vm.at[pl.ds(j * 128, 128)]]
                pltpu.async_copy(src, rows_vm, sem).wait()
                pltpu.sync_copy(rows_vm, dst.at[pl.ds(dst_base + j * 128, 128)])
                return 0

            lax.fori_loop(0, T // 128, g_step, 0)

        invert(rq, idxq_vm, posqf)
        invert(rk, idxk_vm, poskf)
        row0 = h * S + a * T
        gather(qt, idxq_vm, qs, row0)
        gather(kt, idxk_vm, ks, row0)
        gather(vt, idxk_vm, vs, row0)

    return k(rank_q2, rank_k2, qtab, ktab, vtab)


def _sc_unsort(rank_k2, outs, lgs):
    @functools.partial(
        pl.kernel,
        mesh=_SC_MESH,
        compiler_params=pltpu.CompilerParams(needs_layout_passes=False, use_tc_tiling_on_sc=False),
        out_type=[
            jax.ShapeDtypeStruct((NH * S, HS), jnp.float32),
            jax.ShapeDtypeStruct((32, T), jnp.float32),
        ],
        scratch_types=[
            pltpu.VMEM((T,), jnp.int32),
            pltpu.VMEM((T,), jnp.float32),
            pltpu.VMEM((T,), jnp.float32),
            pltpu.VMEM((128, HS), jnp.float32),
            pltpu.SemaphoreType.DMA,
        ],
    )
    def k(rk, osrc, lsrc, odst, ldst, idx_vm, lg_vm, lgo_vm, rows_vm, sem):
        wid = lax.axis_index("s") * 2 + lax.axis_index("c")
        base = wid * T
        pltpu.sync_copy(rk.at[wid], idx_vm)
        pltpu.sync_copy(lsrc.at[pl.ds(base, T)], lg_vm)

        def lg_step(j, _):
            idx = idx_vm[pl.ds(j * 16, 16)]
            lgo_vm[pl.ds(j * 16, 16)] = plsc.load_gather(lg_vm, [idx])
            return 0

        lax.fori_loop(0, T // 16, lg_step, 0)
        pltpu.sync_copy(lgo_vm, ldst.at[wid])

        def mk_global(j, _):
            idx_vm[pl.ds(j * 16, 16)] = idx_vm[pl.ds(j * 16, 16)] + base
            return 0

        lax.fori_loop(0, T // 16, mk_global, 0)

        def g_step(j, _):
            src = osrc.at[idx_vm.at[pl.ds(j * 128, 128)]]
            pltpu.async_copy(src, rows_vm, sem).wait()
            pltpu.sync_copy(rows_vm, odst.at[pl.ds(base + j * 128, 128)])
            return 0

        lax.fori_loop(0, T // 128, g_step, 0)

    return k(rank_k2, outs, lgs)


def kernel(decoder_states, hidden_states, W_qk, W_v):
    ds = decoder_states[0]
    hs = hidden_states[0]
    qkv = _project_qkv(ds, hs, W_qk, W_v)

    rot = jax.random.normal(jax.random.key(HASH_SEED),
                            (NH, HS, NHASH, NB // 2), jnp.float32)
    rot4 = rot.transpose(0, 2, 1, 3)
    ranks = _buckets_ranks(qkv, rot4)[..., 0]
    rank_q2 = ranks[0].reshape(32, T)
    rank_k2 = ranks[1].reshape(32, T)

    posqf, poskf, qs, ks, vs = _sc_sort_gather(
        rank_q2, rank_k2,
        qkv[0].reshape(NH * T, HS),
        qkv[1].reshape(NH * T, HS),
        qkv[2].reshape(NH * T, HS))

    pq = posqf.reshape(NH, S, 1)
    pkl = poskf.reshape(NH, NCH, 1, CHUNK)
    out_s, lg_s = _attention(qs.reshape(NH, S, HS), ks.reshape(NH, S, HS),
                             vs.reshape(NH, S, HS), pq, pkl)

    out_u, lg_u = _sc_unsort(rank_k2, out_s.reshape(NH * S, HS),
                             lg_s.reshape(NH * S))

    final = _combine(out_u.reshape(NH, NHASH, T, HS),
                     lg_u.reshape(NH, NHASH, T, 1))
    return final.reshape(1, T, HIDDEN)
